# baseline (speedup 1.0000x reference)
import functools

import jax
import jax.numpy as jnp
from jax import lax
from jax.experimental import pallas as pl
from jax.experimental.pallas import tpu as pltpu
from jax.experimental.pallas import tpu_sc as plsc

L = 16
NC = 2
NS = 16
NW = NC * NS
C = 64
D = 128
ZP = 10112


@functools.partial(jax.jit, static_argnames=("n_chunks",))
def _decode(z, src, dst, n_chunks):
    k_per_w = n_chunks
    e_per_w = k_per_w * C
    mesh = plsc.VectorSubcoreMesh(core_axis_name="c", subcore_axis_name="s")

    @functools.partial(
        pl.kernel,
        mesh=mesh,
        out_type=jax.ShapeDtypeStruct((NW * e_per_w,), jnp.float32),
        scratch_types=[
            pltpu.VMEM((2, C), jnp.int32),
            pltpu.VMEM((2, C), jnp.int32),
            pltpu.VMEM((C, D), jnp.float32),
            pltpu.VMEM((C, D), jnp.float32),
            pltpu.VMEM((C, D), jnp.float32),
            pltpu.VMEM((C, D), jnp.float32),
            pltpu.VMEM((C,), jnp.float32),
            pltpu.VMEM_SHARED((ZP, D), jnp.float32),
            pltpu.SemaphoreType.DMA,
            pltpu.SemaphoreType.DMA,
            pltpu.SemaphoreType.DMA,
            pltpu.SemaphoreType.DMA,
        ],
    )
    def k(z_hbm, src_hbm, dst_hbm, out_hbm,
          sidx, didx, srows0, drows0, srows1, drows1, obuf, zsh,
          rsem0, rsem1, isem0, isem1):
        wid = lax.axis_index("s") * NC + lax.axis_index("c")
        sid = lax.axis_index("s")
        rows_per_tile = ZP // NS
        pltpu.sync_copy(
            z_hbm.at[pl.ds(sid * rows_per_tile, rows_per_tile)],
            zsh.at[pl.ds(sid * rows_per_tile, rows_per_tile)],
        )
        plsc.subcore_barrier()

        srows = (srows0, srows1)
        drows = (drows0, drows1)
        rsems = (rsem0, rsem1)
        isems = (isem0, isem1)
        lane = lax.iota(jnp.int32, L)
        dists = [1, 2, 4, 8]
        perms = [lane ^ d for d in dists]
        masks = [(lane & d) == 0 for d in dists]

        def idx_issue(c, p):
            pltpu.make_async_copy(src_hbm.at[wid].at[c], sidx.at[p], isems[p]).start()
            pltpu.make_async_copy(dst_hbm.at[wid].at[c], didx.at[p], isems[p]).start()

        def idx_wait(c, p):
            pltpu.make_async_copy(src_hbm.at[wid].at[c], sidx.at[p], isems[p]).wait()
            pltpu.make_async_copy(dst_hbm.at[wid].at[c], didx.at[p], isems[p]).wait()

        def rows_issue(p):
            pltpu.make_async_copy(zsh.at[sidx.at[p]], srows[p], rsems[p]).start()
            pltpu.make_async_copy(zsh.at[didx.at[p]], drows[p], rsems[p]).start()

        def rows_drain(p):
            pltpu.make_async_copy(zsh.at[sidx.at[p]], srows[p], rsems[p]).wait()
            pltpu.make_async_copy(zsh.at[didx.at[p]], drows[p], rsems[p]).wait()

        idx_issue(0, 0)
        idx_wait(0, 0)
        rows_issue(0)
        idx_issue(1, 1)

        def pair_body(cp, carry):
            for b in range(2):
                c = cp * 2 + b
                p = b
                q = 1 - b
                idx_wait(c + 1, q)
                rows_issue(q)
                rows_drain(p)
                idx_issue(c + 2, p)
                sr = srows[p]
                dr = drows[p]

                def group_body(g, carry2):
                    accs = []
                    for u in range(L):
                        e = g * L + u
                        ps = [
                            sr[e, pl.ds(j * L, L)] * dr[e, pl.ds(j * L, L)]
                            for j in range(D // L)
                        ]
                        while len(ps) > 1:
                            ps = [ps[i] + ps[i + 1] for i in range(0, len(ps), 2)]
                        accs.append(ps[0])
                    for lev in range(4):
                        m = masks[lev]
                        pm = perms[lev]
                        nxt = []
                        for k2 in range(0, len(accs), 2):
                            x = accs[k2]
                            y = accs[k2 + 1]
                            xs = jnp.take(x, pm)
                            ys = jnp.take(y, pm)
                            nxt.append(jnp.where(m, x, ys) + jnp.where(m, xs, y))
                        accs = nxt
                    obuf[pl.ds(g * L, L)] = accs[0]
                    return carry2

                lax.fori_loop(0, C // L, group_body, 0, unroll=False)
                pltpu.sync_copy(obuf, out_hbm.at[pl.ds(wid * e_per_w + c * C, C)])
            return carry

        lax.fori_loop(0, k_per_w // 2, pair_body, 0, unroll=False)
        rows_drain(0)
        idx_wait(k_per_w + 1, 1)

    return k(z, src, dst)


def kernel(z, edge_label_index):
    e = edge_label_index.shape[1]
    z = jnp.pad(z, ((0, ZP - z.shape[0]), (0, 0)))
    idx = edge_label_index.astype(jnp.int32)
    per_round = NW * C
    n_chunks = (e + per_round - 1) // per_round
    n_chunks = ((n_chunks + 1) // 2) * 2
    pad = n_chunks * per_round - e
    tail = jnp.broadcast_to(jnp.arange(C, dtype=jnp.int32), (NW, 2, C))
    src = jnp.concatenate(
        [jnp.pad(idx[0], (0, pad)).reshape(NW, n_chunks, C), tail], axis=1)
    dst = jnp.concatenate(
        [jnp.pad(idx[1], (0, pad)).reshape(NW, n_chunks, C), tail], axis=1)
    out = _decode(z, src, dst, n_chunks)
    return out[:e]


# speedup vs baseline: 1.4403x; 1.4403x over previous
import functools

import jax
import jax.numpy as jnp
from jax import lax
from jax.experimental import pallas as pl
from jax.experimental.pallas import tpu as pltpu
from jax.experimental.pallas import tpu_sc as plsc

L = 16
NC = 2
NS = 16
NW = NC * NS
C = 64
D = 128
ZP = 10112


@functools.partial(jax.jit, static_argnames=("n_chunks",))
def _decode(z, src, dst, n_chunks):
    k_per_w = n_chunks
    e_per_w = k_per_w * C
    mesh = plsc.VectorSubcoreMesh(core_axis_name="c", subcore_axis_name="s")

    @functools.partial(
        pl.kernel,
        mesh=mesh,
        out_type=jax.ShapeDtypeStruct((NW * e_per_w,), jnp.float32),
        scratch_types=[
            pltpu.VMEM((2, C), jnp.int32),
            pltpu.VMEM((2, C), jnp.int32),
            pltpu.VMEM((C, D), jnp.float32),
            pltpu.VMEM((C, D), jnp.float32),
            pltpu.VMEM((C, D), jnp.float32),
            pltpu.VMEM((C, D), jnp.float32),
            pltpu.VMEM((C,), jnp.float32),
            pltpu.VMEM_SHARED((ZP, D), jnp.float32),
            pltpu.SemaphoreType.DMA,
            pltpu.SemaphoreType.DMA,
            pltpu.SemaphoreType.DMA,
            pltpu.SemaphoreType.DMA,
        ],
    )
    def k(z_hbm, src_hbm, dst_hbm, out_hbm,
          sidx, didx, srows0, drows0, srows1, drows1, obuf, zsh,
          rsem0, rsem1, isem0, isem1):
        wid = lax.axis_index("s") * NC + lax.axis_index("c")
        sid = lax.axis_index("s")
        rows_per_tile = ZP // NS
        pltpu.sync_copy(
            z_hbm.at[pl.ds(sid * rows_per_tile, rows_per_tile)],
            zsh.at[pl.ds(sid * rows_per_tile, rows_per_tile)],
        )
        plsc.subcore_barrier()

        srows = (srows0, srows1)
        drows = (drows0, drows1)
        rsems = (rsem0, rsem1)
        isems = (isem0, isem1)
        lane = lax.iota(jnp.int32, L)
        dists = [1, 2, 4, 8]
        perms = [lane ^ d for d in dists]
        masks = [(lane & d) == 0 for d in dists]

        def idx_issue(c, p):
            pltpu.make_async_copy(src_hbm.at[wid].at[c], sidx.at[p], isems[p]).start()
            pltpu.make_async_copy(dst_hbm.at[wid].at[c], didx.at[p], isems[p]).start()

        def idx_wait(c, p):
            pltpu.make_async_copy(src_hbm.at[wid].at[c], sidx.at[p], isems[p]).wait()
            pltpu.make_async_copy(dst_hbm.at[wid].at[c], didx.at[p], isems[p]).wait()

        def rows_issue(p):
            pltpu.make_async_copy(zsh.at[sidx.at[p]], srows[p], rsems[p]).start()
            pltpu.make_async_copy(zsh.at[didx.at[p]], drows[p], rsems[p]).start()

        def rows_drain(p):
            pltpu.make_async_copy(zsh.at[sidx.at[p]], srows[p], rsems[p]).wait()
            pltpu.make_async_copy(zsh.at[didx.at[p]], drows[p], rsems[p]).wait()

        idx_issue(0, 0)
        idx_wait(0, 0)
        rows_issue(0)
        idx_issue(1, 1)

        def pair_body(cp, carry):
            for b in range(2):
                c = cp * 2 + b
                p = b
                q = 1 - b
                idx_wait(c + 1, q)
                rows_issue(q)
                rows_drain(p)
                idx_issue(c + 2, p)
                sr = srows[p]
                dr = drows[p]

                def group_body(g, carry2):
                    accs = []
                    for u in range(L):
                        e = g * L + u
                        acc = sr[e, pl.ds(0, L)] * dr[e, pl.ds(0, L)]
                        for j in range(1, D // L):
                            s = sr[e, pl.ds(j * L, L)]
                            t = dr[e, pl.ds(j * L, L)]
                            acc = acc + s * t
                        accs.append(acc)
                    for lev in range(4):
                        m = masks[lev]
                        pm = perms[lev]
                        nxt = []
                        for k2 in range(0, len(accs), 2):
                            x = accs[k2]
                            y = accs[k2 + 1]
                            xs = jnp.take(x, pm)
                            ys = jnp.take(y, pm)
                            nxt.append(jnp.where(m, x, ys) + jnp.where(m, xs, y))
                        accs = nxt
                    obuf[pl.ds(g * L, L)] = accs[0]
                    return carry2

                lax.fori_loop(0, C // L, group_body, 0, unroll=2)
                pltpu.sync_copy(obuf, out_hbm.at[pl.ds(wid * e_per_w + c * C, C)])
            return carry

        lax.fori_loop(0, k_per_w // 2, pair_body, 0, unroll=False)
        rows_drain(0)
        idx_wait(k_per_w + 1, 1)

    return k(z, src, dst)


def kernel(z, edge_label_index):
    e = edge_label_index.shape[1]
    z = jnp.pad(z, ((0, ZP - z.shape[0]), (0, 0)))
    idx = edge_label_index.astype(jnp.int32)
    per_round = NW * C
    n_chunks = (e + per_round - 1) // per_round
    n_chunks = ((n_chunks + 1) // 2) * 2
    pad = n_chunks * per_round - e
    tail = jnp.broadcast_to(jnp.arange(C, dtype=jnp.int32), (NW, 2, C))
    src = jnp.concatenate(
        [jnp.pad(idx[0], (0, pad)).reshape(NW, n_chunks, C), tail], axis=1)
    dst = jnp.concatenate(
        [jnp.pad(idx[1], (0, pad)).reshape(NW, n_chunks, C), tail], axis=1)
    out = _decode(z, src, dst, n_chunks)
    return out[:e]
